# baseline (speedup 1.0000x reference)
import functools

import jax
import jax.numpy as jnp
import numpy as np
from jax import lax
from jax.experimental import pallas as pl
from jax.experimental.pallas import tpu as pltpu
from jax.experimental.pallas import tpu_sc as plsc

_STRIDES = (4, 8, 16, 32, 64)
_NMS_POST = 1000
_IOU_THR = 0.7

_LVL_N = (49152, 12288, 3072, 768, 192)
_LVL_BASE = (0, 49152, 61440, 64512, 65280)
_N_ALL = 65472
_K = 1000

_OUT_BASE = (0, 1008, 2016, 3024, 3792)
_OUT_N = (1008, 1008, 1008, 768, 304)
_KEEP_N = (1000, 1000, 1000, 768, 192)
_M_PAD = 4096
_NMS_R, _NMS_C = 8, 512
_N0 = 1008

_PREV_K = (0, 1000, 2000, 3000, 3768)


def _conv_x(x, w, b, pad):
    y = jax.lax.conv_general_dilated(
        x, w, (1, 1), [(pad, pad), (pad, pad)],
        dimension_numbers=('NCHW', 'OIHW', 'NCHW'))
    return y + b.reshape(1, -1, 1, 1)


def _anchors_for(Hf, Wf, stride):
    ratios = jnp.array([0.5, 1.0, 2.0], dtype=jnp.float32)
    scale = 8.0
    h_r = jnp.sqrt(ratios)
    w_r = 1.0 / h_r
    ws = stride * scale * w_r
    hs = stride * scale * h_r
    base = jnp.stack([-0.5 * ws, -0.5 * hs, 0.5 * ws, 0.5 * hs], axis=1)
    sx = jnp.arange(Wf, dtype=jnp.float32) * stride
    sy = jnp.arange(Hf, dtype=jnp.float32) * stride
    yy, xx = jnp.meshgrid(sy, sx, indexing='ij')
    shifts = jnp.stack([xx.ravel(), yy.ravel(), xx.ravel(), yy.ravel()], axis=1)
    return (shifts[:, None, :] + base[None, :, :]).reshape(-1, 4)


def _decode(anchors, deltas, max_h, max_w):
    px = (anchors[:, 0] + anchors[:, 2]) * 0.5
    py = (anchors[:, 1] + anchors[:, 3]) * 0.5
    pw = anchors[:, 2] - anchors[:, 0]
    ph = anchors[:, 3] - anchors[:, 1]
    dx, dy, dw, dh = deltas[:, 0], deltas[:, 1], deltas[:, 2], deltas[:, 3]
    max_ratio = float(np.abs(np.log(16.0 / 1000.0)))
    dw = jnp.clip(dw, -max_ratio, max_ratio)
    dh = jnp.clip(dh, -max_ratio, max_ratio)
    gw = pw * jnp.exp(dw)
    gh = ph * jnp.exp(dh)
    gx = px + pw * dx
    gy = py + ph * dy
    x1 = jnp.clip(gx - 0.5 * gw, 0.0, max_w)
    y1 = jnp.clip(gy - 0.5 * gh, 0.0, max_h)
    x2 = jnp.clip(gx + 0.5 * gw, 0.0, max_w)
    y2 = jnp.clip(gy + 0.5 * gh, 0.0, max_h)
    return jnp.stack([x1, y1, x2, y2], axis=1)


def _thresh_body(s0_ref, s1_ref, s2_ref, out_ref):
    def kth(bits):
        def it(_, lohi):
            lo, hi = lohi
            mid = lo + (hi - lo) // 2
            cnt = jnp.sum((bits >= mid).astype(jnp.int32))
            pred = cnt >= _K
            return (jnp.where(pred, mid, lo), jnp.where(pred, hi, mid))
        lo, _ = lax.fori_loop(0, 31, it, (jnp.int32(1), jnp.int32(0x3F800001)))
        return lax.bitcast_convert_type(lo, jnp.float32)

    t0 = kth(lax.bitcast_convert_type(s0_ref[...], jnp.int32))
    t1 = kth(lax.bitcast_convert_type(s1_ref[...], jnp.int32))
    t2 = kth(lax.bitcast_convert_type(s2_ref[...], jnp.int32))
    lane = jax.lax.broadcasted_iota(jnp.int32, (8, 128), 1)
    tiny = jnp.float32(1e-35)
    row = jnp.where(lane == 0, t0,
                    jnp.where(lane == 1, t1,
                              jnp.where(lane == 2, t2, tiny)))
    out_ref[...] = row


def _thresholds(s_all):
    s0 = s_all[0:49152].reshape(8, 6144)
    s1 = s_all[49152:61440].reshape(8, 1536)
    s2 = s_all[61440:64512].reshape(8, 384)
    out = pl.pallas_call(
        _thresh_body,
        out_shape=jax.ShapeDtypeStruct((8, 128), jnp.float32),
    )(s0, s1, s2)
    tiny = jnp.float32(1e-35)
    thr5 = jnp.stack([out[0, 0], out[0, 1], out[0, 2], tiny, tiny])
    return jnp.broadcast_to(thr5.reshape(5, 1), (5, 16)).astype(jnp.float32)


def _sc_gather_kernel(s_hbm, x1_hbm, y1_hbm, x2_hbm, y2_hbm, idx_hbm,
                      s_out, x1_out, y1_out, x2_out, y2_out,
                      idx_v, g_s, g_x1, g_y1, g_x2, g_y2, sem):
    c = lax.axis_index("c")
    s_id = lax.axis_index("s")
    wid = s_id * 2 + c
    base = wid * 128
    pltpu.sync_copy(idx_hbm.at[pl.ds(base, 128)], idx_v)
    pltpu.async_copy(s_hbm.at[idx_v], g_s, sem).wait()
    pltpu.async_copy(x1_hbm.at[idx_v], g_x1, sem).wait()
    pltpu.async_copy(y1_hbm.at[idx_v], g_y1, sem).wait()
    pltpu.async_copy(x2_hbm.at[idx_v], g_x2, sem).wait()
    pltpu.async_copy(y2_hbm.at[idx_v], g_y2, sem).wait()
    pltpu.sync_copy(g_s, s_out.at[pl.ds(base, 128)])
    pltpu.sync_copy(g_x1, x1_out.at[pl.ds(base, 128)])
    pltpu.sync_copy(g_y1, y1_out.at[pl.ds(base, 128)])
    pltpu.sync_copy(g_x2, x2_out.at[pl.ds(base, 128)])
    pltpu.sync_copy(g_y2, y2_out.at[pl.ds(base, 128)])


def _sc_gather(s_all, x1, y1, x2, y2, idx):
    fvec = jax.ShapeDtypeStruct((_M_PAD,), jnp.float32)
    mesh = plsc.VectorSubcoreMesh(core_axis_name="c", subcore_axis_name="s")
    kern = functools.partial(
        pl.kernel,
        out_type=[fvec, fvec, fvec, fvec, fvec],
        mesh=mesh,
        scratch_types=[pltpu.VMEM((128,), jnp.int32)]
                      + [pltpu.VMEM((128,), jnp.float32)] * 5
                      + [pltpu.SemaphoreType.DMA],
    )(_sc_gather_kernel)
    return kern(s_all, x1, y1, x2, y2, idx)


def _nms_body(s_ref, x1_ref, y1_ref, x2_ref, y2_ref, out_ref):
    shape = (_NMS_R, _NMS_C)
    s = s_ref[...]
    x1 = x1_ref[...]
    y1 = y1_ref[...]
    x2 = x2_ref[...]
    y2 = y2_ref[...]
    flat = (jax.lax.broadcasted_iota(jnp.int32, shape, 0) * _NMS_C
            + jax.lax.broadcasted_iota(jnp.int32, shape, 1))
    hole = (((flat >= 1000) & (flat < 1008))
            | ((flat >= 2008) & (flat < 2016))
            | ((flat >= 3016) & (flat < 3024))
            | (flat >= 3984))
    s = jnp.where(hole, jnp.float32(-jnp.inf), s)
    areas = (x2 - x1) * (y2 - y1)
    neg = jnp.float32(-jnp.inf)
    big = jnp.int32(2 ** 30)

    mask0 = flat < _N0
    s0 = jnp.where(mask0, s, neg)
    m0 = jnp.max(s0)
    j0 = jnp.min(jnp.where(mask0 & (s0 == m0), flat, big))

    lane4 = jax.lax.broadcasted_iota(jnp.int32, (1, 4), 1)

    def body(i, sw):
        m = jnp.max(sw)
        j = jnp.min(jnp.where(sw == m, flat, big))
        jj = jnp.where(m == neg, j0, j)
        pick = flat == jj
        xb1 = jnp.max(jnp.where(pick, x1, neg))
        yb1 = jnp.max(jnp.where(pick, y1, neg))
        xb2 = jnp.max(jnp.where(pick, x2, neg))
        yb2 = jnp.max(jnp.where(pick, y2, neg))
        ab = (xb2 - xb1) * (yb2 - yb1)
        iw = jnp.maximum(jnp.minimum(x2, xb2) - jnp.maximum(x1, xb1), 0.0)
        ih = jnp.maximum(jnp.minimum(y2, yb2) - jnp.maximum(y1, yb1), 0.0)
        inter = iw * ih
        iou = inter / (areas + ab - inter + jnp.float32(1e-9))
        sw = jnp.where((iou > jnp.float32(_IOU_THR)) | pick, neg, sw)
        row = jnp.where(lane4 == 0, xb1,
                        jnp.where(lane4 == 1, yb1,
                                  jnp.where(lane4 == 2, xb2, yb2)))
        out_ref[pl.ds(i, 1), :] = row
        return sw

    jax.lax.fori_loop(0, _NMS_POST, body, s)


def _nms_pallas(s, x1, y1, x2, y2):
    return pl.pallas_call(
        _nms_body,
        out_shape=jax.ShapeDtypeStruct((_NMS_POST, 4), jnp.float32),
    )(s.reshape(_NMS_R, _NMS_C), x1.reshape(_NMS_R, _NMS_C),
      y1.reshape(_NMS_R, _NMS_C), x2.reshape(_NMS_R, _NMS_C),
      y2.reshape(_NMS_R, _NMS_C))


def kernel(feat0, feat1, feat2, feat3, feat4, x, W_conv, b_conv,
           W_cls, b_cls, W_reg, b_reg):
    img_h = float(x.shape[2])
    img_w = float(x.shape[3])
    feats = (feat0, feat1, feat2, feat3, feat4)
    sc_l, x1_l, y1_l, x2_l, y2_l = [], [], [], [], []
    for feat, stride in zip(feats, _STRIDES):
        t = jax.nn.relu(_conv_x(feat, W_conv, b_conv, 1))
        cls = _conv_x(t, W_cls, b_cls, 0)
        reg = _conv_x(t, W_reg, b_reg, 0)
        Hf, Wf = feat.shape[2], feat.shape[3]
        anchors = _anchors_for(Hf, Wf, float(stride))
        scores = jax.nn.sigmoid(cls.transpose(0, 2, 3, 1).reshape(-1))
        deltas = reg.transpose(0, 2, 3, 1).reshape(-1, 4)
        props = _decode(anchors, deltas, img_h, img_w)
        sc_l.append(scores)
        x1_l.append(props[:, 0])
        y1_l.append(props[:, 1])
        x2_l.append(props[:, 2])
        y2_l.append(props[:, 3])
    s_all = jnp.concatenate(sc_l)
    x1a = jnp.concatenate(x1_l)
    y1a = jnp.concatenate(y1_l)
    x2a = jnp.concatenate(x2_l)
    y2a = jnp.concatenate(y2_l)
    thr = _thresholds(s_all)
    thr_elem = jnp.concatenate(
        [jnp.full((_LVL_N[l],), thr[l, 0], jnp.float32) for l in range(5)])
    mask = s_all >= thr_elem
    csum = jnp.cumsum(mask.astype(jnp.int32))
    adjust = jnp.concatenate(
        [jnp.full((_LVL_N[l],), _OUT_BASE[l] - _PREV_K[l], jnp.int32)
         for l in range(5)])
    target = jnp.where(mask, csum - 1 + adjust, _M_PAD)
    src_iota = jnp.arange(_N_ALL, dtype=jnp.int32)
    idx = jnp.zeros((_M_PAD,), jnp.int32).at[target].set(src_iota, mode="drop")
    cs, cx1, cy1, cx2, cy2 = _sc_gather(s_all, x1a, y1a, x2a, y2a, idx)
    kept = _nms_pallas(cs, cx1, cy1, cx2, cy2)
    return kept[None]


# speedup vs baseline: 1.0013x; 1.0013x over previous
import functools

import jax
import jax.numpy as jnp
import numpy as np
from jax import lax
from jax.experimental import pallas as pl
from jax.experimental.pallas import tpu as pltpu
from jax.experimental.pallas import tpu_sc as plsc

_STRIDES = (4, 8, 16, 32, 64)
_NMS_POST = 1000
_IOU_THR = 0.7

_LVL_N = (49152, 12288, 3072, 768, 192)
_LVL_BASE = (0, 49152, 61440, 64512, 65280)
_N_ALL = 65472
_K = 1000

_OUT_BASE = (0, 1008, 2016, 3024, 3792)
_OUT_N = (1008, 1008, 1008, 768, 304)
_KEEP_N = (1000, 1000, 1000, 768, 192)
_M_PAD = 4096
_NMS_R, _NMS_C = 8, 512
_N0 = 1008

_PREV_K = (0, 1000, 2000, 3000, 3768)


def _conv_x(x, w, b, pad):
    y = jax.lax.conv_general_dilated(
        x, w, (1, 1), [(pad, pad), (pad, pad)],
        dimension_numbers=('NCHW', 'OIHW', 'NCHW'))
    return y + b.reshape(1, -1, 1, 1)


def _anchors_for(Hf, Wf, stride):
    ratios = jnp.array([0.5, 1.0, 2.0], dtype=jnp.float32)
    scale = 8.0
    h_r = jnp.sqrt(ratios)
    w_r = 1.0 / h_r
    ws = stride * scale * w_r
    hs = stride * scale * h_r
    base = jnp.stack([-0.5 * ws, -0.5 * hs, 0.5 * ws, 0.5 * hs], axis=1)
    sx = jnp.arange(Wf, dtype=jnp.float32) * stride
    sy = jnp.arange(Hf, dtype=jnp.float32) * stride
    yy, xx = jnp.meshgrid(sy, sx, indexing='ij')
    shifts = jnp.stack([xx.ravel(), yy.ravel(), xx.ravel(), yy.ravel()], axis=1)
    return (shifts[:, None, :] + base[None, :, :]).reshape(-1, 4)


def _decode(anchors, deltas, max_h, max_w):
    px = (anchors[:, 0] + anchors[:, 2]) * 0.5
    py = (anchors[:, 1] + anchors[:, 3]) * 0.5
    pw = anchors[:, 2] - anchors[:, 0]
    ph = anchors[:, 3] - anchors[:, 1]
    dx, dy, dw, dh = deltas[:, 0], deltas[:, 1], deltas[:, 2], deltas[:, 3]
    max_ratio = float(np.abs(np.log(16.0 / 1000.0)))
    dw = jnp.clip(dw, -max_ratio, max_ratio)
    dh = jnp.clip(dh, -max_ratio, max_ratio)
    gw = pw * jnp.exp(dw)
    gh = ph * jnp.exp(dh)
    gx = px + pw * dx
    gy = py + ph * dy
    x1 = jnp.clip(gx - 0.5 * gw, 0.0, max_w)
    y1 = jnp.clip(gy - 0.5 * gh, 0.0, max_h)
    x2 = jnp.clip(gx + 0.5 * gw, 0.0, max_w)
    y2 = jnp.clip(gy + 0.5 * gh, 0.0, max_h)
    return jnp.stack([x1, y1, x2, y2], axis=1)


def _thresh_body(s0_ref, s1_ref, s2_ref, out_ref):
    def kth(bits):
        def it(_, lohi):
            lo, hi = lohi
            mid = lo + (hi - lo) // 2
            cnt = jnp.sum((bits >= mid).astype(jnp.int32))
            pred = cnt >= _K
            return (jnp.where(pred, mid, lo), jnp.where(pred, hi, mid))
        lo, _ = lax.fori_loop(0, 31, it, (jnp.int32(1), jnp.int32(0x3F800001)))
        return lax.bitcast_convert_type(lo, jnp.float32)

    t0 = kth(lax.bitcast_convert_type(s0_ref[...], jnp.int32))
    t1 = kth(lax.bitcast_convert_type(s1_ref[...], jnp.int32))
    t2 = kth(lax.bitcast_convert_type(s2_ref[...], jnp.int32))
    lane = jax.lax.broadcasted_iota(jnp.int32, (8, 128), 1)
    tiny = jnp.float32(1e-35)
    row = jnp.where(lane == 0, t0,
                    jnp.where(lane == 1, t1,
                              jnp.where(lane == 2, t2, tiny)))
    out_ref[...] = row


def _thresholds(s_all):
    s0 = s_all[0:49152].reshape(8, 6144)
    s1 = s_all[49152:61440].reshape(8, 1536)
    s2 = s_all[61440:64512].reshape(8, 384)
    out = pl.pallas_call(
        _thresh_body,
        out_shape=jax.ShapeDtypeStruct((8, 128), jnp.float32),
    )(s0, s1, s2)
    tiny = jnp.float32(1e-35)
    thr5 = jnp.stack([out[0, 0], out[0, 1], out[0, 2], tiny, tiny])
    return jnp.broadcast_to(thr5.reshape(5, 1), (5, 16)).astype(jnp.float32)


def _sc_gather_kernel(s_hbm, x1_hbm, y1_hbm, x2_hbm, y2_hbm, idx_hbm,
                      s_out, x1_out, y1_out, x2_out, y2_out,
                      idx_v, g_s, g_x1, g_y1, g_x2, g_y2, sem):
    c = lax.axis_index("c")
    s_id = lax.axis_index("s")
    wid = s_id * 2 + c
    base = wid * 128
    pltpu.sync_copy(idx_hbm.at[pl.ds(base, 128)], idx_v)
    cps = [pltpu.async_copy(s_hbm.at[idx_v], g_s, sem),
           pltpu.async_copy(x1_hbm.at[idx_v], g_x1, sem),
           pltpu.async_copy(y1_hbm.at[idx_v], g_y1, sem),
           pltpu.async_copy(x2_hbm.at[idx_v], g_x2, sem),
           pltpu.async_copy(y2_hbm.at[idx_v], g_y2, sem)]
    for cp in cps:
        cp.wait()
    pltpu.sync_copy(g_s, s_out.at[pl.ds(base, 128)])
    pltpu.sync_copy(g_x1, x1_out.at[pl.ds(base, 128)])
    pltpu.sync_copy(g_y1, y1_out.at[pl.ds(base, 128)])
    pltpu.sync_copy(g_x2, x2_out.at[pl.ds(base, 128)])
    pltpu.sync_copy(g_y2, y2_out.at[pl.ds(base, 128)])


def _sc_gather(s_all, x1, y1, x2, y2, idx):
    fvec = jax.ShapeDtypeStruct((_M_PAD,), jnp.float32)
    mesh = plsc.VectorSubcoreMesh(core_axis_name="c", subcore_axis_name="s")
    kern = functools.partial(
        pl.kernel,
        out_type=[fvec, fvec, fvec, fvec, fvec],
        mesh=mesh,
        scratch_types=[pltpu.VMEM((128,), jnp.int32)]
                      + [pltpu.VMEM((128,), jnp.float32)] * 5
                      + [pltpu.SemaphoreType.DMA],
    )(_sc_gather_kernel)
    return kern(s_all, x1, y1, x2, y2, idx)


def _nms_body(s_ref, x1_ref, y1_ref, x2_ref, y2_ref, out_ref):
    shape = (_NMS_R, _NMS_C)
    s = s_ref[...]
    x1 = x1_ref[...]
    y1 = y1_ref[...]
    x2 = x2_ref[...]
    y2 = y2_ref[...]
    flat = (jax.lax.broadcasted_iota(jnp.int32, shape, 0) * _NMS_C
            + jax.lax.broadcasted_iota(jnp.int32, shape, 1))
    hole = (((flat >= 1000) & (flat < 1008))
            | ((flat >= 2008) & (flat < 2016))
            | ((flat >= 3016) & (flat < 3024))
            | (flat >= 3984))
    s = jnp.where(hole, jnp.float32(-jnp.inf), s)
    areas = (x2 - x1) * (y2 - y1)
    neg = jnp.float32(-jnp.inf)
    big = jnp.int32(2 ** 30)

    mask0 = flat < _N0
    s0 = jnp.where(mask0, s, neg)
    m0 = jnp.max(s0)
    j0 = jnp.min(jnp.where(mask0 & (s0 == m0), flat, big))

    lane4 = jax.lax.broadcasted_iota(jnp.int32, (1, 4), 1)

    def body(i, sw):
        m = jnp.max(sw)
        j = jnp.min(jnp.where(sw == m, flat, big))
        jj = jnp.where(m == neg, j0, j)
        pick = flat == jj
        xb1 = jnp.max(jnp.where(pick, x1, neg))
        yb1 = jnp.max(jnp.where(pick, y1, neg))
        xb2 = jnp.max(jnp.where(pick, x2, neg))
        yb2 = jnp.max(jnp.where(pick, y2, neg))
        ab = (xb2 - xb1) * (yb2 - yb1)
        iw = jnp.maximum(jnp.minimum(x2, xb2) - jnp.maximum(x1, xb1), 0.0)
        ih = jnp.maximum(jnp.minimum(y2, yb2) - jnp.maximum(y1, yb1), 0.0)
        inter = iw * ih
        iou = inter / (areas + ab - inter + jnp.float32(1e-9))
        sw = jnp.where((iou > jnp.float32(_IOU_THR)) | pick, neg, sw)
        row = jnp.where(lane4 == 0, xb1,
                        jnp.where(lane4 == 1, yb1,
                                  jnp.where(lane4 == 2, xb2, yb2)))
        out_ref[pl.ds(i, 1), :] = row
        return sw

    jax.lax.fori_loop(0, _NMS_POST, body, s)


def _nms_pallas(s, x1, y1, x2, y2):
    return pl.pallas_call(
        _nms_body,
        out_shape=jax.ShapeDtypeStruct((_NMS_POST, 4), jnp.float32),
    )(s.reshape(_NMS_R, _NMS_C), x1.reshape(_NMS_R, _NMS_C),
      y1.reshape(_NMS_R, _NMS_C), x2.reshape(_NMS_R, _NMS_C),
      y2.reshape(_NMS_R, _NMS_C))


def kernel(feat0, feat1, feat2, feat3, feat4, x, W_conv, b_conv,
           W_cls, b_cls, W_reg, b_reg):
    img_h = float(x.shape[2])
    img_w = float(x.shape[3])
    feats = (feat0, feat1, feat2, feat3, feat4)
    sc_l, x1_l, y1_l, x2_l, y2_l = [], [], [], [], []
    for feat, stride in zip(feats, _STRIDES):
        t = jax.nn.relu(_conv_x(feat, W_conv, b_conv, 1))
        cls = _conv_x(t, W_cls, b_cls, 0)
        reg = _conv_x(t, W_reg, b_reg, 0)
        Hf, Wf = feat.shape[2], feat.shape[3]
        anchors = _anchors_for(Hf, Wf, float(stride))
        scores = jax.nn.sigmoid(cls.transpose(0, 2, 3, 1).reshape(-1))
        deltas = reg.transpose(0, 2, 3, 1).reshape(-1, 4)
        props = _decode(anchors, deltas, img_h, img_w)
        sc_l.append(scores)
        x1_l.append(props[:, 0])
        y1_l.append(props[:, 1])
        x2_l.append(props[:, 2])
        y2_l.append(props[:, 3])
    s_all = jnp.concatenate(sc_l)
    x1a = jnp.concatenate(x1_l)
    y1a = jnp.concatenate(y1_l)
    x2a = jnp.concatenate(x2_l)
    y2a = jnp.concatenate(y2_l)
    thr = _thresholds(s_all)
    thr_elem = jnp.concatenate(
        [jnp.full((_LVL_N[l],), thr[l, 0], jnp.float32) for l in range(5)])
    mask = s_all >= thr_elem
    csum = jnp.cumsum(mask.astype(jnp.int32))
    adjust = jnp.concatenate(
        [jnp.full((_LVL_N[l],), _OUT_BASE[l] - _PREV_K[l], jnp.int32)
         for l in range(5)])
    target = jnp.where(mask, csum - 1 + adjust, _M_PAD)
    src_iota = jnp.arange(_N_ALL, dtype=jnp.int32)
    idx = jnp.zeros((_M_PAD,), jnp.int32).at[target].set(src_iota, mode="drop")
    cs, cx1, cy1, cx2, cy2 = _sc_gather(s_all, x1a, y1a, x2a, y2a, idx)
    kept = _nms_pallas(cs, cx1, cy1, cx2, cy2)
    return kept[None]


# speedup vs baseline: 1.0081x; 1.0068x over previous
import functools

import jax
import jax.numpy as jnp
import numpy as np
from jax import lax
from jax.experimental import pallas as pl
from jax.experimental.pallas import tpu as pltpu
from jax.experimental.pallas import tpu_sc as plsc

_STRIDES = (4, 8, 16, 32, 64)
_NMS_POST = 1000
_IOU_THR = 0.7

_LVL_N = (49152, 12288, 3072, 768, 192)
_LVL_BASE = (0, 49152, 61440, 64512, 65280)
_N_ALL = 65472
_K = 1000

_OUT_BASE = (0, 1008, 2016, 3024, 3792)
_OUT_N = (1008, 1008, 1008, 768, 304)
_KEEP_N = (1000, 1000, 1000, 768, 192)
_M_PAD = 4096
_NMS_R, _NMS_C = 8, 512
_N0 = 1008

_PREV_K = (0, 1000, 2000, 3000, 3768)


def _conv_x(x, w, b, pad):
    y = jax.lax.conv_general_dilated(
        x, w, (1, 1), [(pad, pad), (pad, pad)],
        dimension_numbers=('NCHW', 'OIHW', 'NCHW'))
    return y + b.reshape(1, -1, 1, 1)


def _anchors_for(Hf, Wf, stride):
    ratios = jnp.array([0.5, 1.0, 2.0], dtype=jnp.float32)
    scale = 8.0
    h_r = jnp.sqrt(ratios)
    w_r = 1.0 / h_r
    ws = stride * scale * w_r
    hs = stride * scale * h_r
    base = jnp.stack([-0.5 * ws, -0.5 * hs, 0.5 * ws, 0.5 * hs], axis=1)
    sx = jnp.arange(Wf, dtype=jnp.float32) * stride
    sy = jnp.arange(Hf, dtype=jnp.float32) * stride
    yy, xx = jnp.meshgrid(sy, sx, indexing='ij')
    shifts = jnp.stack([xx.ravel(), yy.ravel(), xx.ravel(), yy.ravel()], axis=1)
    return (shifts[:, None, :] + base[None, :, :]).reshape(-1, 4)


def _decode(anchors, deltas, max_h, max_w):
    px = (anchors[:, 0] + anchors[:, 2]) * 0.5
    py = (anchors[:, 1] + anchors[:, 3]) * 0.5
    pw = anchors[:, 2] - anchors[:, 0]
    ph = anchors[:, 3] - anchors[:, 1]
    dx, dy, dw, dh = deltas[:, 0], deltas[:, 1], deltas[:, 2], deltas[:, 3]
    max_ratio = float(np.abs(np.log(16.0 / 1000.0)))
    dw = jnp.clip(dw, -max_ratio, max_ratio)
    dh = jnp.clip(dh, -max_ratio, max_ratio)
    gw = pw * jnp.exp(dw)
    gh = ph * jnp.exp(dh)
    gx = px + pw * dx
    gy = py + ph * dy
    x1 = jnp.clip(gx - 0.5 * gw, 0.0, max_w)
    y1 = jnp.clip(gy - 0.5 * gh, 0.0, max_h)
    x2 = jnp.clip(gx + 0.5 * gw, 0.0, max_w)
    y2 = jnp.clip(gy + 0.5 * gh, 0.0, max_h)
    return jnp.stack([x1, y1, x2, y2], axis=1)


def _thresh_body(s0_ref, s1_ref, s2_ref, out_ref):
    b0 = lax.bitcast_convert_type(s0_ref[...], jnp.int32)
    b1 = lax.bitcast_convert_type(s1_ref[...], jnp.int32)
    b2 = lax.bitcast_convert_type(s2_ref[...], jnp.int32)

    def it(_, st):
        lo0, hi0, lo1, hi1, lo2, hi2 = st
        m0 = lo0 + (hi0 - lo0) // 2
        m1 = lo1 + (hi1 - lo1) // 2
        m2 = lo2 + (hi2 - lo2) // 2
        c0 = jnp.sum((b0 >= m0).astype(jnp.int32))
        c1 = jnp.sum((b1 >= m1).astype(jnp.int32))
        c2 = jnp.sum((b2 >= m2).astype(jnp.int32))
        p0 = c0 >= _K
        p1 = c1 >= _K
        p2 = c2 >= _K
        return (jnp.where(p0, m0, lo0), jnp.where(p0, hi0, m0),
                jnp.where(p1, m1, lo1), jnp.where(p1, hi1, m1),
                jnp.where(p2, m2, lo2), jnp.where(p2, hi2, m2))

    one = jnp.int32(1)
    top = jnp.int32(0x3F800001)
    lo0, _, lo1, _, lo2, _ = lax.fori_loop(
        0, 31, it, (one, top, one, top, one, top))
    t0 = lax.bitcast_convert_type(lo0, jnp.float32)
    t1 = lax.bitcast_convert_type(lo1, jnp.float32)
    t2 = lax.bitcast_convert_type(lo2, jnp.float32)
    lane = jax.lax.broadcasted_iota(jnp.int32, (8, 128), 1)
    tiny = jnp.float32(1e-35)
    row = jnp.where(lane == 0, t0,
                    jnp.where(lane == 1, t1,
                              jnp.where(lane == 2, t2, tiny)))
    out_ref[...] = row


def _thresholds(s_all):
    s0 = s_all[0:49152].reshape(8, 6144)
    s1 = s_all[49152:61440].reshape(8, 1536)
    s2 = s_all[61440:64512].reshape(8, 384)
    out = pl.pallas_call(
        _thresh_body,
        out_shape=jax.ShapeDtypeStruct((8, 128), jnp.float32),
    )(s0, s1, s2)
    tiny = jnp.float32(1e-35)
    thr5 = jnp.stack([out[0, 0], out[0, 1], out[0, 2], tiny, tiny])
    return jnp.broadcast_to(thr5.reshape(5, 1), (5, 16)).astype(jnp.float32)


def _sc_gather_kernel(s_hbm, x1_hbm, y1_hbm, x2_hbm, y2_hbm, idx_hbm,
                      s_out, x1_out, y1_out, x2_out, y2_out,
                      idx_v, g_s, g_x1, g_y1, g_x2, g_y2, sem):
    c = lax.axis_index("c")
    s_id = lax.axis_index("s")
    wid = s_id * 2 + c
    base = wid * 128
    pltpu.sync_copy(idx_hbm.at[pl.ds(base, 128)], idx_v)
    cps = [pltpu.async_copy(s_hbm.at[idx_v], g_s, sem),
           pltpu.async_copy(x1_hbm.at[idx_v], g_x1, sem),
           pltpu.async_copy(y1_hbm.at[idx_v], g_y1, sem),
           pltpu.async_copy(x2_hbm.at[idx_v], g_x2, sem),
           pltpu.async_copy(y2_hbm.at[idx_v], g_y2, sem)]
    for cp in cps:
        cp.wait()
    pltpu.sync_copy(g_s, s_out.at[pl.ds(base, 128)])
    pltpu.sync_copy(g_x1, x1_out.at[pl.ds(base, 128)])
    pltpu.sync_copy(g_y1, y1_out.at[pl.ds(base, 128)])
    pltpu.sync_copy(g_x2, x2_out.at[pl.ds(base, 128)])
    pltpu.sync_copy(g_y2, y2_out.at[pl.ds(base, 128)])


def _sc_gather(s_all, x1, y1, x2, y2, idx):
    fvec = jax.ShapeDtypeStruct((_M_PAD,), jnp.float32)
    mesh = plsc.VectorSubcoreMesh(core_axis_name="c", subcore_axis_name="s")
    kern = functools.partial(
        pl.kernel,
        out_type=[fvec, fvec, fvec, fvec, fvec],
        mesh=mesh,
        scratch_types=[pltpu.VMEM((128,), jnp.int32)]
                      + [pltpu.VMEM((128,), jnp.float32)] * 5
                      + [pltpu.SemaphoreType.DMA],
    )(_sc_gather_kernel)
    return kern(s_all, x1, y1, x2, y2, idx)


def _nms_body(s_ref, x1_ref, y1_ref, x2_ref, y2_ref, out_ref):
    shape = (_NMS_R, _NMS_C)
    s = s_ref[...]
    x1 = x1_ref[...]
    y1 = y1_ref[...]
    x2 = x2_ref[...]
    y2 = y2_ref[...]
    flat = (jax.lax.broadcasted_iota(jnp.int32, shape, 0) * _NMS_C
            + jax.lax.broadcasted_iota(jnp.int32, shape, 1))
    hole = (((flat >= 1000) & (flat < 1008))
            | ((flat >= 2008) & (flat < 2016))
            | ((flat >= 3016) & (flat < 3024))
            | (flat >= 3984))
    s = jnp.where(hole, jnp.float32(-jnp.inf), s)
    areas = (x2 - x1) * (y2 - y1)
    neg = jnp.float32(-jnp.inf)
    big = jnp.int32(2 ** 30)

    mask0 = flat < _N0
    s0 = jnp.where(mask0, s, neg)
    m0 = jnp.max(s0)
    j0 = jnp.min(jnp.where(mask0 & (s0 == m0), flat, big))

    lane4 = jax.lax.broadcasted_iota(jnp.int32, (1, 4), 1)

    def body(i, sw):
        m = jnp.max(sw)
        j = jnp.min(jnp.where(sw == m, flat, big))
        jj = jnp.where(m == neg, j0, j)
        pick = flat == jj
        xb1 = jnp.max(jnp.where(pick, x1, neg))
        yb1 = jnp.max(jnp.where(pick, y1, neg))
        xb2 = jnp.max(jnp.where(pick, x2, neg))
        yb2 = jnp.max(jnp.where(pick, y2, neg))
        ab = (xb2 - xb1) * (yb2 - yb1)
        iw = jnp.maximum(jnp.minimum(x2, xb2) - jnp.maximum(x1, xb1), 0.0)
        ih = jnp.maximum(jnp.minimum(y2, yb2) - jnp.maximum(y1, yb1), 0.0)
        inter = iw * ih
        iou = inter / (areas + ab - inter + jnp.float32(1e-9))
        sw = jnp.where((iou > jnp.float32(_IOU_THR)) | pick, neg, sw)
        row = jnp.where(lane4 == 0, xb1,
                        jnp.where(lane4 == 1, yb1,
                                  jnp.where(lane4 == 2, xb2, yb2)))
        out_ref[pl.ds(i, 1), :] = row
        return sw

    jax.lax.fori_loop(0, _NMS_POST, body, s)


def _nms_pallas(s, x1, y1, x2, y2):
    return pl.pallas_call(
        _nms_body,
        out_shape=jax.ShapeDtypeStruct((_NMS_POST, 4), jnp.float32),
    )(s.reshape(_NMS_R, _NMS_C), x1.reshape(_NMS_R, _NMS_C),
      y1.reshape(_NMS_R, _NMS_C), x2.reshape(_NMS_R, _NMS_C),
      y2.reshape(_NMS_R, _NMS_C))


def kernel(feat0, feat1, feat2, feat3, feat4, x, W_conv, b_conv,
           W_cls, b_cls, W_reg, b_reg):
    img_h = float(x.shape[2])
    img_w = float(x.shape[3])
    feats = (feat0, feat1, feat2, feat3, feat4)
    sc_l, x1_l, y1_l, x2_l, y2_l = [], [], [], [], []
    for feat, stride in zip(feats, _STRIDES):
        t = jax.nn.relu(_conv_x(feat, W_conv, b_conv, 1))
        cls = _conv_x(t, W_cls, b_cls, 0)
        reg = _conv_x(t, W_reg, b_reg, 0)
        Hf, Wf = feat.shape[2], feat.shape[3]
        anchors = _anchors_for(Hf, Wf, float(stride))
        scores = jax.nn.sigmoid(cls.transpose(0, 2, 3, 1).reshape(-1))
        deltas = reg.transpose(0, 2, 3, 1).reshape(-1, 4)
        props = _decode(anchors, deltas, img_h, img_w)
        sc_l.append(scores)
        x1_l.append(props[:, 0])
        y1_l.append(props[:, 1])
        x2_l.append(props[:, 2])
        y2_l.append(props[:, 3])
    s_all = jnp.concatenate(sc_l)
    x1a = jnp.concatenate(x1_l)
    y1a = jnp.concatenate(y1_l)
    x2a = jnp.concatenate(x2_l)
    y2a = jnp.concatenate(y2_l)
    thr = _thresholds(s_all)
    thr_elem = jnp.concatenate(
        [jnp.full((_LVL_N[l],), thr[l, 0], jnp.float32) for l in range(5)])
    mask = s_all >= thr_elem
    csum = jnp.cumsum(mask.astype(jnp.int32))
    adjust = jnp.concatenate(
        [jnp.full((_LVL_N[l],), _OUT_BASE[l] - _PREV_K[l], jnp.int32)
         for l in range(5)])
    target = jnp.where(mask, csum - 1 + adjust, _M_PAD)
    src_iota = jnp.arange(_N_ALL, dtype=jnp.int32)
    idx = jnp.zeros((_M_PAD,), jnp.int32).at[target].set(src_iota, mode="drop")
    cs, cx1, cy1, cx2, cy2 = _sc_gather(s_all, x1a, y1a, x2a, y2a, idx)
    kept = _nms_pallas(cs, cx1, cy1, cx2, cy2)
    return kept[None]
